# Initial kernel scaffold; baseline (speedup 1.0000x reference)
import jax
import jax.numpy as jnp
from jax.experimental import pallas as pl


def kernel(x, edge_index, W1a, b1a, ga, ba, W2a, b2a, W1b, b1b, gb, bb, W2b, b2b):
    raise NotImplementedError("write your pallas kernel here")


# speedup vs baseline: 11.4646x; 11.4646x over previous
import functools

import jax
import jax.numpy as jnp
from jax import lax
from jax.experimental import pallas as pl
from jax.experimental.pallas import tpu as pltpu
from jax.experimental.pallas import tpu_sc as plsc

N = 10000
D = 128
E = 320000
NC = 2
NS = 16
NW = NC * NS
EPW = E // NW
K = 80
NCHUNK = 128
EPWP = NCHUNK * K
NHALF = NCHUNK // 2
NPAD = 10240
RPT = NPAD // NS
WBC = 160

_sc_mesh = plsc.VectorSubcoreMesh(core_axis_name="c", subcore_axis_name="s")


@functools.partial(
    pl.kernel,
    mesh=_sc_mesh,
    out_type=jax.ShapeDtypeStruct((NC * NPAD, D), jnp.float32),
    scratch_types=[
        pltpu.VMEM((NHALF, K), jnp.int32),
        pltpu.VMEM((NHALF, K), jnp.int32),
        pltpu.VMEM((K, D), jnp.float32),
        pltpu.VMEM((K, D), jnp.float32),
        pltpu.VMEM((K, D), jnp.float32),
        pltpu.VMEM_SHARED((NPAD, D), jnp.float32),
        pltpu.SemaphoreType.DMA,
        pltpu.SemaphoreType.DMA,
        pltpu.SemaphoreType.DMA,
    ],
)
def _sc_agg(x_hbm, src_hbm, dst_hbm, zeros_hbm, out_hbm,
            src_v, dst_v, rows0_v, rows1_v, rows2_v, acc_sh, sem0, sem1, sem2):
    c = lax.axis_index("c")
    s = lax.axis_index("s")
    wid = s * NC + c

    for j in range(RPT // WBC):
        pltpu.sync_copy(zeros_hbm, acc_sh.at[pl.ds(s * RPT + j * WBC, WBC)])
    plsc.subcore_barrier()

    rows = (rows0_v, rows1_v, rows2_v)
    sems = (sem0, sem1, sem2)

    def start(g, b):
        pltpu.async_copy(x_hbm.at[src_v.at[g]], rows[b], sems[b])

    def finish(g, b):
        pltpu.make_async_copy(x_hbm.at[src_v.at[g]], rows[b], sems[b]).wait()
        pltpu.sync_copy(rows[b], acc_sh.at[dst_v.at[g]], add=True)

    for h in range(NCHUNK // NHALF):
        pltpu.sync_copy(src_hbm.at[wid, pl.ds(h * NHALF, NHALF)], src_v)
        pltpu.sync_copy(dst_hbm.at[wid, pl.ds(h * NHALF, NHALF)], dst_v)
        start(0, 0)
        start(1, 1)

        def body(i, carry):
            g0 = 3 * i
            start(g0 + 2, 2)
            finish(g0, 0)
            start(g0 + 3, 0)
            finish(g0 + 1, 1)
            start(g0 + 4, 1)
            finish(g0 + 2, 2)
            return carry

        lax.fori_loop(0, 20, body, 0)
        start(62, 2)
        finish(60, 0)
        finish(61, 1)
        finish(62, 2)
        start(63, 0)
        finish(63, 0)
    plsc.subcore_barrier()

    for j in range(RPT // WBC):
        r = s * RPT + j * WBC
        pltpu.sync_copy(acc_sh.at[pl.ds(r, WBC)],
                        out_hbm.at[pl.ds(c * NPAD + r, WBC)])


def _tc_mlp_body(x_ref, p_ref, w1_ref, b1_ref, g_ref, be_ref, w2_ref, b2_ref,
                 o_ref):
    h = x_ref[...] + p_ref[0:N, :] + p_ref[NPAD:NPAD + N, :]
    y = jnp.dot(h, w1_ref[...], preferred_element_type=jnp.float32) + b1_ref[...]
    mean = jnp.mean(y, axis=0, keepdims=True)
    yc = y - mean
    var = jnp.mean(yc * yc, axis=0, keepdims=True)
    yn = yc * lax.rsqrt(var + 1e-5) * g_ref[...] + be_ref[...]
    yr = jnp.maximum(yn, 0.0)
    z = jnp.dot(yr, w2_ref[...], preferred_element_type=jnp.float32) + b2_ref[...]
    o_ref[...] = jnp.maximum(z, 0.0)


def _tc_mlp(x, p, w1, b1, g, be, w2, b2):
    return pl.pallas_call(
        _tc_mlp_body,
        out_shape=jax.ShapeDtypeStruct((N, D), jnp.float32),
    )(x, p, w1, b1.reshape(1, D), g.reshape(1, D), be.reshape(1, D), w2,
      b2.reshape(1, D))


def kernel(x, edge_index, W1a, b1a, ga, ba, W2a, b2a, W1b, b1b, gb, bb, W2b,
           b2b):
    srcw = edge_index[0].astype(jnp.int32).reshape(NW, EPW)
    dstw = edge_index[1].astype(jnp.int32).reshape(NW, EPW)
    npad = EPWP - EPW
    pad_src = (jnp.arange(NW, dtype=jnp.int32)[:, None] * 601
               + jnp.arange(npad, dtype=jnp.int32)[None, :] * 41) % N
    src = jnp.concatenate([srcw, pad_src], axis=1).reshape(NW, NCHUNK, K)
    pad_dst = jnp.broadcast_to(
        N + jnp.arange(npad, dtype=jnp.int32) % (NPAD - N), (NW, npad))
    dst = jnp.concatenate([dstw, pad_dst], axis=1).reshape(NW, NCHUNK, K)
    zeros = jnp.zeros((WBC, D), jnp.float32)
    p1 = _sc_agg(x, src, dst, zeros)
    h1 = _tc_mlp(x, p1, W1a, b1a, ga, ba, W2a, b2a)
    p2 = _sc_agg(h1, src, dst, zeros)
    h2 = _tc_mlp(h1, p2, W1b, b1b, gb, bb, W2b, b2b)
    return (h1, h2)


# speedup vs baseline: 11.6949x; 1.0201x over previous
import functools

import jax
import jax.numpy as jnp
from jax import lax
from jax.experimental import pallas as pl
from jax.experimental.pallas import tpu as pltpu
from jax.experimental.pallas import tpu_sc as plsc

N = 10000
D = 128
E = 320000
NC = 2
NS = 16
NW = NC * NS
EPW = E // NW
K = 80
NCHUNK = 128
EPWP = NCHUNK * K
NHALF = NCHUNK // 2
NPAD = 10240
RPT = NPAD // NS
WBC = RPT

_sc_mesh = plsc.VectorSubcoreMesh(core_axis_name="c", subcore_axis_name="s")


@functools.partial(
    pl.kernel,
    mesh=_sc_mesh,
    out_type=jax.ShapeDtypeStruct((NC * NPAD, D), jnp.float32),
    scratch_types=[
        pltpu.VMEM((NHALF, K), jnp.int32),
        pltpu.VMEM((NHALF, K), jnp.int32),
        pltpu.VMEM((K, D), jnp.float32),
        pltpu.VMEM((K, D), jnp.float32),
        pltpu.VMEM((K, D), jnp.float32),
        pltpu.VMEM_SHARED((NPAD, D), jnp.float32),
        pltpu.SemaphoreType.DMA,
        pltpu.SemaphoreType.DMA,
        pltpu.SemaphoreType.DMA,
    ],
)
def _sc_agg(x_hbm, src_hbm, dst_hbm, zeros_hbm, out_hbm,
            src_v, dst_v, rows0_v, rows1_v, rows2_v, acc_sh, sem0, sem1, sem2):
    c = lax.axis_index("c")
    s = lax.axis_index("s")
    wid = s * NC + c

    for j in range(RPT // WBC):
        pltpu.sync_copy(zeros_hbm, acc_sh.at[pl.ds(s * RPT + j * WBC, WBC)])
    plsc.subcore_barrier()

    rows = (rows0_v, rows1_v, rows2_v)
    sems = (sem0, sem1, sem2)

    def start(g, b):
        pltpu.async_copy(x_hbm.at[src_v.at[g]], rows[b], sems[b])

    def finish(g, b):
        pltpu.make_async_copy(x_hbm.at[src_v.at[g]], rows[b], sems[b]).wait()
        pltpu.sync_copy(rows[b], acc_sh.at[dst_v.at[g]], add=True)

    for h in range(NCHUNK // NHALF):
        pltpu.sync_copy(src_hbm.at[wid, pl.ds(h * NHALF, NHALF)], src_v)
        pltpu.sync_copy(dst_hbm.at[wid, pl.ds(h * NHALF, NHALF)], dst_v)
        start(0, 0)
        start(1, 1)

        def body(i, carry):
            g0 = 3 * i
            start(g0 + 2, 2)
            finish(g0, 0)
            start(g0 + 3, 0)
            finish(g0 + 1, 1)
            start(g0 + 4, 1)
            finish(g0 + 2, 2)
            return carry

        lax.fori_loop(0, 20, body, 0)
        start(62, 2)
        finish(60, 0)
        finish(61, 1)
        finish(62, 2)
        start(63, 0)
        finish(63, 0)
    plsc.subcore_barrier()

    for j in range(RPT // WBC):
        r = s * RPT + j * WBC
        pltpu.sync_copy(acc_sh.at[pl.ds(r, WBC)],
                        out_hbm.at[pl.ds(c * NPAD + r, WBC)])


def _tc_mlp_body(x_ref, p_ref, w1_ref, b1_ref, g_ref, be_ref, w2_ref, b2_ref,
                 o_ref):
    h = x_ref[...] + p_ref[0:N, :] + p_ref[NPAD:NPAD + N, :]
    y = jnp.dot(h, w1_ref[...], preferred_element_type=jnp.float32) + b1_ref[...]
    mean = jnp.mean(y, axis=0, keepdims=True)
    yc = y - mean
    var = jnp.mean(yc * yc, axis=0, keepdims=True)
    yn = yc * lax.rsqrt(var + 1e-5) * g_ref[...] + be_ref[...]
    yr = jnp.maximum(yn, 0.0)
    z = jnp.dot(yr, w2_ref[...], preferred_element_type=jnp.float32) + b2_ref[...]
    o_ref[...] = jnp.maximum(z, 0.0)


def _tc_mlp(x, p, w1, b1, g, be, w2, b2):
    return pl.pallas_call(
        _tc_mlp_body,
        out_shape=jax.ShapeDtypeStruct((N, D), jnp.float32),
    )(x, p, w1, b1.reshape(1, D), g.reshape(1, D), be.reshape(1, D), w2,
      b2.reshape(1, D))


def kernel(x, edge_index, W1a, b1a, ga, ba, W2a, b2a, W1b, b1b, gb, bb, W2b,
           b2b):
    srcw = edge_index[0].astype(jnp.int32).reshape(NW, EPW)
    dstw = edge_index[1].astype(jnp.int32).reshape(NW, EPW)
    npad = EPWP - EPW
    pad_src = (jnp.arange(NW, dtype=jnp.int32)[:, None] * 601
               + jnp.arange(npad, dtype=jnp.int32)[None, :] * 41) % N
    src = jnp.concatenate([srcw, pad_src], axis=1).reshape(NW, NCHUNK, K)
    pad_dst = jnp.broadcast_to(
        N + jnp.arange(npad, dtype=jnp.int32) % (NPAD - N), (NW, npad))
    dst = jnp.concatenate([dstw, pad_dst], axis=1).reshape(NW, NCHUNK, K)
    zeros = jnp.zeros((WBC, D), jnp.float32)
    p1 = _sc_agg(x, src, dst, zeros)
    h1 = _tc_mlp(x, p1, W1a, b1a, ga, ba, W2a, b2a)
    p2 = _sc_agg(h1, src, dst, zeros)
    h2 = _tc_mlp(h1, p2, W1b, b1b, gb, bb, W2b, b2b)
    return (h1, h2)
